# Initial kernel scaffold; baseline (speedup 1.0000x reference)
import jax
import jax.numpy as jnp
from jax.experimental import pallas as pl


def kernel(logits, targets, ious):
    raise NotImplementedError("write your pallas kernel here")


# speedup vs baseline: 21.2711x; 21.2711x over previous
import functools

import jax
import jax.numpy as jnp
from jax.experimental import pallas as pl
from jax.experimental.pallas import tpu as pltpu

_TI = 256
_TJ = 1024
_SENT_LO = jnp.float32(-1e30)
_SENT_HI = jnp.float32(1e30)
_TH_Z = -4.0


def _bg_body(b_ref, a_ref, rank_ref, bce_ref):
    j = pl.program_id(1)

    @pl.when(j == 0)
    def _():
        rank_ref[...] = jnp.zeros_like(rank_ref)
        bce_ref[...] = jnp.zeros_like(bce_ref)

    a_max = a_ref[0, _TJ - 1]
    b_min = b_ref[0, 0]

    @pl.when(a_max - b_min > _TH_Z)
    def _():
        z = a_ref[...] - b_ref[...]
        gt = z > _TH_Z
        s = jax.nn.sigmoid(z)
        sp = jnp.maximum(z, 0.0) - jnp.log(jnp.maximum(s, 1.0 - s))
        bce = jnp.minimum(sp, 100.0)
        rank_ref[...] += jnp.sum(jnp.where(gt, s, 0.0), axis=1, keepdims=True)
        bce_ref[...] += jnp.sum(jnp.where(gt, bce, 0.0), axis=1, keepdims=True)


def _fg_body(b_ref, ui_ref, a_ref, uc_ref, rank_ref, bce_ref):
    j = pl.program_id(1)

    @pl.when(j == 0)
    def _():
        rank_ref[...] = jnp.zeros_like(rank_ref)
        bce_ref[...] = jnp.zeros_like(bce_ref)

    a_max = a_ref[0, _TJ - 1]
    b_min = b_ref[0, 0]

    @pl.when(a_max - b_min > _TH_Z)
    def _():
        z = a_ref[...] - b_ref[...]
        gt = z > _TH_Z
        s = jax.nn.sigmoid(z)
        sp = jnp.maximum(z, 0.0) - jnp.log(jnp.maximum(s, 1.0 - s))
        bce = jnp.minimum(sp, 100.0)
        fp = gt & (uc_ref[...] < ui_ref[...])
        rank_ref[...] += jnp.sum(jnp.where(gt, s, 0.0), axis=1, keepdims=True)
        bce_ref[...] += jnp.sum(jnp.where(fp, bce, 0.0), axis=1, keepdims=True)


def _pair_call(body, n_rows, grid_cols, operands):
    grid = (n_rows // _TI, grid_cols // _TJ)
    row_spec = pl.BlockSpec((_TI, 1), lambda i, j: (i, 0))
    col_spec = pl.BlockSpec((1, _TJ), lambda i, j: (0, j))
    out_spec = pl.BlockSpec((_TI, 1), lambda i, j: (i, 0))
    in_specs = [row_spec if op.shape[1] == 1 else col_spec for op in operands]
    return pl.pallas_call(
        body,
        out_shape=[jax.ShapeDtypeStruct((n_rows, 1), jnp.float32)] * 2,
        grid=grid,
        in_specs=in_specs,
        out_specs=[out_spec, out_spec],
        compiler_params=pltpu.CompilerParams(
            dimension_semantics=("parallel", "arbitrary")),
    )(*operands)


def kernel(logits, targets, ious):
    lamb = 4.0
    loss_weight = 1.0
    th = -4.0 / lamb
    n = logits.shape[0]
    f = ious.shape[0]

    logits = logits.astype(jnp.float32)
    ious = ious.astype(jnp.float32)

    fg_mask = targets == 1
    fg_idx = jnp.nonzero(fg_mask, size=f, fill_value=0)[0]
    x_fg = logits[fg_idx] * lamb

    order = jnp.argsort(x_fg)
    xr_s = x_fg[order]
    ur_s = ious[order]

    p_min = jnp.min(x_fg) * (1.0 / lamb)
    cutoff = (p_min + th) * lamb
    a_bg_full = jnp.where((targets == 0) & (logits * lamb >= cutoff),
                          logits * lamb, _SENT_LO)
    a_bg = jnp.sort(a_bg_full)
    nb_pad = ((n + _TJ - 1) // _TJ) * _TJ
    a_bg = jnp.concatenate(
        [jnp.full((nb_pad - n,), _SENT_LO, jnp.float32), a_bg])

    fa_pad = ((f + _TJ - 1) // _TJ) * _TJ
    pad_a = fa_pad - f
    a_fg = jnp.concatenate([jnp.full((pad_a,), _SENT_LO, jnp.float32), xr_s])
    u_fg = jnp.concatenate([jnp.zeros((pad_a,), jnp.float32), ur_s])

    f_pad = ((f + _TI - 1) // _TI) * _TI
    pad_r = f_pad - f
    xr = jnp.concatenate([xr_s, jnp.full((pad_r,), _SENT_HI, jnp.float32)])
    ur = jnp.concatenate([ur_s, jnp.zeros((pad_r,), jnp.float32)])
    real = jnp.concatenate([jnp.ones((f,), jnp.float32),
                            jnp.zeros((pad_r,), jnp.float32)])
    nb = f_pad // _TI
    k = jnp.arange(nb)
    blk = jnp.where(k % 2 == 0, k // 2, nb - 1 - k // 2)
    perm = (blk[:, None] * _TI + jnp.arange(_TI)[None, :]).reshape(-1)
    xr, ur, real = xr[perm], ur[perm], real[perm]

    xr2 = xr.reshape(f_pad, 1)
    ur2 = ur.reshape(f_pad, 1)

    rank_b, bce_b = _pair_call(_bg_body, f_pad, nb_pad,
                               (xr2, a_bg.reshape(1, nb_pad)))
    rank_a, bce_a = _pair_call(_fg_body, f_pad, fa_pad,
                               (xr2, ur2, a_fg.reshape(1, fa_pad),
                                u_fg.reshape(1, fa_pad)))

    rank = rank_a[:, 0] + rank_b[:, 0]
    bces = bce_a[:, 0] + bce_b[:, 0]

    valid = (real > 0) & (bces > 0)
    loss_i = jnp.where(valid, bces * ur / jnp.where(valid, rank, 1.0), 0.0)
    n_valid = jnp.sum(valid.astype(jnp.float32))
    return loss_weight * jnp.sum(loss_i) / jnp.maximum(n_valid, 1.0) / lamb


# speedup vs baseline: 32.7466x; 1.5395x over previous
import functools

import jax
import jax.numpy as jnp
from jax.experimental import pallas as pl
from jax.experimental.pallas import tpu as pltpu

_TI = 256
_TJ = 2048
_SENT_LO = -1e30
_SENT_HI = 1e30
_TH_Z = -4.0


def _rank_bce(z):
    s = 0.5 * jnp.tanh(z * 0.5) + 0.5
    sp = jnp.maximum(z, 0.0) - jnp.log(jnp.maximum(s, 1.0 - s))
    return s, jnp.minimum(sp, 100.0)


def _bg_body(b_ref, a_ref, rank_ref, bce_ref):
    j = pl.program_id(1)

    @pl.when(j == 0)
    def _():
        rank_ref[...] = jnp.zeros_like(rank_ref)
        bce_ref[...] = jnp.zeros_like(bce_ref)

    a_min = a_ref[0, 0]
    a_max = a_ref[0, _TJ - 1]
    b_min = b_ref[0, 0]
    b_max = b_ref[_TI - 1, 0]
    any_hit = a_max - b_min > _TH_Z
    all_hit = a_min - b_max > _TH_Z

    @pl.when(all_hit)
    def _():
        z = a_ref[...] - b_ref[...]
        s, bce = _rank_bce(z)
        rank_ref[...] += jnp.sum(s, axis=1, keepdims=True)
        bce_ref[...] += jnp.sum(bce, axis=1, keepdims=True)

    @pl.when(any_hit & ~all_hit)
    def _():
        z = a_ref[...] - b_ref[...]
        gt = z > _TH_Z
        s, bce = _rank_bce(z)
        rank_ref[...] += jnp.sum(jnp.where(gt, s, 0.0), axis=1, keepdims=True)
        bce_ref[...] += jnp.sum(jnp.where(gt, bce, 0.0), axis=1, keepdims=True)


def _fg_body(b_ref, ui_ref, a_ref, uc_ref, rank_ref, bce_ref):
    j = pl.program_id(1)

    @pl.when(j == 0)
    def _():
        rank_ref[...] = jnp.zeros_like(rank_ref)
        bce_ref[...] = jnp.zeros_like(bce_ref)

    a_min = a_ref[0, 0]
    a_max = a_ref[0, _TJ - 1]
    b_min = b_ref[0, 0]
    b_max = b_ref[_TI - 1, 0]
    any_hit = a_max - b_min > _TH_Z
    all_hit = a_min - b_max > _TH_Z

    @pl.when(all_hit)
    def _():
        z = a_ref[...] - b_ref[...]
        s, bce = _rank_bce(z)
        lt = uc_ref[...] < ui_ref[...]
        rank_ref[...] += jnp.sum(s, axis=1, keepdims=True)
        bce_ref[...] += jnp.sum(jnp.where(lt, bce, 0.0), axis=1, keepdims=True)

    @pl.when(any_hit & ~all_hit)
    def _():
        z = a_ref[...] - b_ref[...]
        gt = z > _TH_Z
        s, bce = _rank_bce(z)
        fp = gt & (uc_ref[...] < ui_ref[...])
        rank_ref[...] += jnp.sum(jnp.where(gt, s, 0.0), axis=1, keepdims=True)
        bce_ref[...] += jnp.sum(jnp.where(fp, bce, 0.0), axis=1, keepdims=True)


def _pair_call(body, n_rows, grid_cols, operands):
    grid = (n_rows // _TI, grid_cols // _TJ)
    row_spec = pl.BlockSpec((_TI, 1), lambda i, j: (i, 0))
    col_spec = pl.BlockSpec((1, _TJ), lambda i, j: (0, j))
    out_spec = pl.BlockSpec((_TI, 1), lambda i, j: (i, 0))
    in_specs = [row_spec if op.shape[1] == 1 else col_spec for op in operands]
    return pl.pallas_call(
        body,
        out_shape=[jax.ShapeDtypeStruct((n_rows, 1), jnp.float32)] * 2,
        grid=grid,
        in_specs=in_specs,
        out_specs=[out_spec, out_spec],
        compiler_params=pltpu.CompilerParams(
            dimension_semantics=("parallel", "arbitrary")),
    )(*operands)


def kernel(logits, targets, ious):
    lamb = 4.0
    loss_weight = 1.0
    th = -4.0 / lamb
    n = logits.shape[0]
    f = ious.shape[0]

    logits = logits.astype(jnp.float32)
    ious = ious.astype(jnp.float32)

    fg_mask = targets == 1
    fg_idx = jnp.nonzero(fg_mask, size=f, fill_value=0)[0]
    x_fg = logits[fg_idx] * lamb

    order = jnp.argsort(x_fg)
    xr_s = x_fg[order]
    ur_s = ious[order]

    p_min = jnp.min(x_fg) * (1.0 / lamb)
    cutoff = (p_min + th) * lamb
    a_bg_full = jnp.where((targets == 0) & (logits * lamb >= cutoff),
                          logits * lamb, _SENT_LO)
    a_bg = jnp.sort(a_bg_full)
    nb_pad = ((n + _TJ - 1) // _TJ) * _TJ
    a_bg = jnp.concatenate(
        [jnp.full((nb_pad - n,), _SENT_LO, jnp.float32), a_bg])

    fa_pad = ((f + _TJ - 1) // _TJ) * _TJ
    pad_a = fa_pad - f
    a_fg = jnp.concatenate([jnp.full((pad_a,), _SENT_LO, jnp.float32), xr_s])
    u_fg = jnp.concatenate([jnp.zeros((pad_a,), jnp.float32), ur_s])

    f_pad = ((f + _TI - 1) // _TI) * _TI
    pad_r = f_pad - f
    xr = jnp.concatenate([xr_s, jnp.full((pad_r,), _SENT_HI, jnp.float32)])
    ur = jnp.concatenate([ur_s, jnp.zeros((pad_r,), jnp.float32)])
    real = jnp.concatenate([jnp.ones((f,), jnp.float32),
                            jnp.zeros((pad_r,), jnp.float32)])
    nb = f_pad // _TI
    k = jnp.arange(nb)
    blk = jnp.where(k % 2 == 0, k // 2, nb - 1 - k // 2)
    perm = (blk[:, None] * _TI + jnp.arange(_TI)[None, :]).reshape(-1)
    xr, ur, real = xr[perm], ur[perm], real[perm]

    xr2 = xr.reshape(f_pad, 1)
    ur2 = ur.reshape(f_pad, 1)

    rank_b, bce_b = _pair_call(_bg_body, f_pad, nb_pad,
                               (xr2, a_bg.reshape(1, nb_pad)))
    rank_a, bce_a = _pair_call(_fg_body, f_pad, fa_pad,
                               (xr2, ur2, a_fg.reshape(1, fa_pad),
                                u_fg.reshape(1, fa_pad)))

    rank = rank_a[:, 0] + rank_b[:, 0]
    bces = bce_a[:, 0] + bce_b[:, 0]

    valid = (real > 0) & (bces > 0)
    loss_i = jnp.where(valid, bces * ur / jnp.where(valid, rank, 1.0), 0.0)
    n_valid = jnp.sum(valid.astype(jnp.float32))
    return loss_weight * jnp.sum(loss_i) / jnp.maximum(n_valid, 1.0) / lamb


# speedup vs baseline: 36.6861x; 1.1203x over previous
import functools

import jax
import jax.numpy as jnp
from jax.experimental import pallas as pl
from jax.experimental.pallas import tpu as pltpu

_TI = 256
_TJ = 2048
_SENT_LO = -1e30
_SENT_HI = 1e30
_TH_Z = -4.0


def _rank_bce(z):
    s = 0.5 * jnp.tanh(z * 0.5) + 0.5
    sp = jnp.maximum(z, 0.0) - jnp.log(jnp.maximum(s, 1.0 - s))
    return s, jnp.minimum(sp, 100.0)


def _bg_body(amin_s, amax_s, bmin_s, bmax_s,
             b_ref, a_ref, rank_ref, bce_ref):
    i = pl.program_id(0)
    j = pl.program_id(1)

    @pl.when(j == 0)
    def _():
        rank_ref[...] = jnp.zeros_like(rank_ref)
        bce_ref[...] = jnp.zeros_like(bce_ref)

    any_hit = amax_s[j] - bmin_s[i] > _TH_Z
    all_hit = amin_s[j] - bmax_s[i] > _TH_Z

    @pl.when(all_hit)
    def _():
        z = a_ref[...] - b_ref[...]
        s, bce = _rank_bce(z)
        rank_ref[...] += jnp.sum(s, axis=1, keepdims=True)
        bce_ref[...] += jnp.sum(bce, axis=1, keepdims=True)

    @pl.when(any_hit & ~all_hit)
    def _():
        z = a_ref[...] - b_ref[...]
        gt = z > _TH_Z
        s, bce = _rank_bce(z)
        rank_ref[...] += jnp.sum(jnp.where(gt, s, 0.0), axis=1, keepdims=True)
        bce_ref[...] += jnp.sum(jnp.where(gt, bce, 0.0), axis=1, keepdims=True)


def _fg_body(amin_s, amax_s, bmin_s, bmax_s,
             b_ref, ui_ref, a_ref, uc_ref, rank_ref, bce_ref):
    i = pl.program_id(0)
    j = pl.program_id(1)

    @pl.when(j == 0)
    def _():
        rank_ref[...] = jnp.zeros_like(rank_ref)
        bce_ref[...] = jnp.zeros_like(bce_ref)

    any_hit = amax_s[j] - bmin_s[i] > _TH_Z
    all_hit = amin_s[j] - bmax_s[i] > _TH_Z

    @pl.when(all_hit)
    def _():
        z = a_ref[...] - b_ref[...]
        s, bce = _rank_bce(z)
        lt = uc_ref[...] < ui_ref[...]
        rank_ref[...] += jnp.sum(s, axis=1, keepdims=True)
        bce_ref[...] += jnp.sum(jnp.where(lt, bce, 0.0), axis=1, keepdims=True)

    @pl.when(any_hit & ~all_hit)
    def _():
        z = a_ref[...] - b_ref[...]
        gt = z > _TH_Z
        s, bce = _rank_bce(z)
        fp = gt & (uc_ref[...] < ui_ref[...])
        rank_ref[...] += jnp.sum(jnp.where(gt, s, 0.0), axis=1, keepdims=True)
        bce_ref[...] += jnp.sum(jnp.where(fp, bce, 0.0), axis=1, keepdims=True)


def _pair_call(body, n_rows, grid_cols, scalars, operands):
    grid = (n_rows // _TI, grid_cols // _TJ)
    row_spec = pl.BlockSpec((_TI, 1), lambda i, j, *_: (i, 0))
    col_spec = pl.BlockSpec((1, _TJ), lambda i, j, *_: (0, j))
    out_spec = pl.BlockSpec((_TI, 1), lambda i, j, *_: (i, 0))
    in_specs = [row_spec if op.shape[1] == 1 else col_spec for op in operands]
    return pl.pallas_call(
        body,
        out_shape=[jax.ShapeDtypeStruct((n_rows, 1), jnp.float32)] * 2,
        grid_spec=pltpu.PrefetchScalarGridSpec(
            num_scalar_prefetch=len(scalars),
            grid=grid,
            in_specs=in_specs,
            out_specs=[out_spec, out_spec],
        ),
        compiler_params=pltpu.CompilerParams(
            dimension_semantics=("parallel", "arbitrary")),
    )(*scalars, *operands)


def kernel(logits, targets, ious):
    lamb = 4.0
    loss_weight = 1.0
    th = -4.0 / lamb
    n = logits.shape[0]
    f = ious.shape[0]

    logits = logits.astype(jnp.float32)
    ious = ious.astype(jnp.float32)

    fg_mask = targets == 1
    fg_idx = jnp.nonzero(fg_mask, size=f, fill_value=0)[0]
    x_fg = logits[fg_idx] * lamb

    order = jnp.argsort(x_fg)
    xr_s = x_fg[order]
    ur_s = ious[order]

    p_min = jnp.min(x_fg) * (1.0 / lamb)
    cutoff = (p_min + th) * lamb
    a_bg_full = jnp.where((targets == 0) & (logits * lamb >= cutoff),
                          logits * lamb, _SENT_LO)
    a_bg = jnp.sort(a_bg_full)
    nb_pad = ((n + _TJ - 1) // _TJ) * _TJ
    a_bg = jnp.concatenate(
        [jnp.full((nb_pad - n,), _SENT_LO, jnp.float32), a_bg])

    fa_pad = ((f + _TJ - 1) // _TJ) * _TJ
    pad_a = fa_pad - f
    a_fg = jnp.concatenate([jnp.full((pad_a,), _SENT_LO, jnp.float32), xr_s])
    u_fg = jnp.concatenate([jnp.zeros((pad_a,), jnp.float32), ur_s])

    f_pad = ((f + _TI - 1) // _TI) * _TI
    pad_r = f_pad - f
    xr = jnp.concatenate([xr_s, jnp.full((pad_r,), _SENT_HI, jnp.float32)])
    ur = jnp.concatenate([ur_s, jnp.zeros((pad_r,), jnp.float32)])
    real = jnp.concatenate([jnp.ones((f,), jnp.float32),
                            jnp.zeros((pad_r,), jnp.float32)])
    nb = f_pad // _TI
    k = jnp.arange(nb)
    blk = jnp.where(k % 2 == 0, k // 2, nb - 1 - k // 2)
    perm = (blk[:, None] * _TI + jnp.arange(_TI)[None, :]).reshape(-1)
    xr, ur, real = xr[perm], ur[perm], real[perm]

    b_blk = xr.reshape(-1, _TI)
    bmin_s, bmax_s = b_blk[:, 0], b_blk[:, _TI - 1]
    abg_blk = a_bg.reshape(-1, _TJ)
    abg_min, abg_max = abg_blk[:, 0], abg_blk[:, _TJ - 1]
    afg_blk = a_fg.reshape(-1, _TJ)
    afg_min, afg_max = afg_blk[:, 0], afg_blk[:, _TJ - 1]

    xr2 = xr.reshape(f_pad, 1)
    ur2 = ur.reshape(f_pad, 1)

    rank_b, bce_b = _pair_call(_bg_body, f_pad, nb_pad,
                               (abg_min, abg_max, bmin_s, bmax_s),
                               (xr2, a_bg.reshape(1, nb_pad)))
    rank_a, bce_a = _pair_call(_fg_body, f_pad, fa_pad,
                               (afg_min, afg_max, bmin_s, bmax_s),
                               (xr2, ur2, a_fg.reshape(1, fa_pad),
                                u_fg.reshape(1, fa_pad)))

    rank = rank_a[:, 0] + rank_b[:, 0]
    bces = bce_a[:, 0] + bce_b[:, 0]

    valid = (real > 0) & (bces > 0)
    loss_i = jnp.where(valid, bces * ur / jnp.where(valid, rank, 1.0), 0.0)
    n_valid = jnp.sum(valid.astype(jnp.float32))
    return loss_weight * jnp.sum(loss_i) / jnp.maximum(n_valid, 1.0) / lamb
